# baseline (speedup 1.0000x reference)
import jax
import jax.numpy as jnp
from jax.experimental import pallas as pl
from jax.experimental.pallas import tpu as pltpu


def _router_kernel(x_ref, w_ref, b_ref, logits_ref, probs_ref):
    x = x_ref[...]
    w = w_ref[...]
    logits = jax.lax.dot_general(
        x, w,
        dimension_numbers=(((1,), (1,)), ((), ())),
        preferred_element_type=jnp.float32,
        precision=jax.lax.Precision.HIGHEST,
    ) + b_ref[...]
    logits_ref[...] = logits
    m = jnp.max(logits, axis=-1, keepdims=True)
    e = jnp.exp(logits - m)
    probs_ref[...] = e / jnp.sum(e, axis=-1, keepdims=True)


def kernel(input, W, b):
    n, d = input.shape
    num_experts = W.shape[0]
    tile = 1024
    grid = (n // tile,)
    b2 = b.reshape(1, num_experts)
    logits, probs = pl.pallas_call(
        _router_kernel,
        grid=grid,
        in_specs=[
            pl.BlockSpec((tile, d), lambda i: (i, 0)),
            pl.BlockSpec((num_experts, d), lambda i: (0, 0)),
            pl.BlockSpec((1, num_experts), lambda i: (0, 0)),
        ],
        out_specs=[
            pl.BlockSpec((tile, num_experts), lambda i: (i, 0)),
            pl.BlockSpec((tile, num_experts), lambda i: (i, 0)),
        ],
        out_shape=[
            jax.ShapeDtypeStruct((n, num_experts), jnp.float32),
            jax.ShapeDtypeStruct((n, num_experts), jnp.float32),
        ],
        compiler_params=pltpu.CompilerParams(
            dimension_semantics=("parallel",),
        ),
    )(input, W, b2)
    return (logits, probs)


# speedup vs baseline: 1.5305x; 1.5305x over previous
import jax
import jax.numpy as jnp
from jax.experimental import pallas as pl
from jax.experimental.pallas import tpu as pltpu


def _router_kernel(x_ref, w_ref, b_ref, logits_ref, probs_ref):
    x = x_ref[...]
    w = w_ref[...]
    logits = jax.lax.dot_general(
        x, w,
        dimension_numbers=(((1,), (1,)), ((), ())),
        preferred_element_type=jnp.float32,
        precision=jax.lax.Precision.DEFAULT,
    ) + b_ref[...]
    logits_ref[...] = logits
    m = jnp.max(logits, axis=-1, keepdims=True)
    e = jnp.exp(logits - m)
    probs_ref[...] = e / jnp.sum(e, axis=-1, keepdims=True)


def kernel(input, W, b):
    n, d = input.shape
    num_experts = W.shape[0]
    tile = 1024
    grid = (n // tile,)
    b2 = b.reshape(1, num_experts)
    logits, probs = pl.pallas_call(
        _router_kernel,
        grid=grid,
        in_specs=[
            pl.BlockSpec((tile, d), lambda i: (i, 0)),
            pl.BlockSpec((num_experts, d), lambda i: (0, 0)),
            pl.BlockSpec((1, num_experts), lambda i: (0, 0)),
        ],
        out_specs=[
            pl.BlockSpec((tile, num_experts), lambda i: (i, 0)),
            pl.BlockSpec((tile, num_experts), lambda i: (i, 0)),
        ],
        out_shape=[
            jax.ShapeDtypeStruct((n, num_experts), jnp.float32),
            jax.ShapeDtypeStruct((n, num_experts), jnp.float32),
        ],
        compiler_params=pltpu.CompilerParams(
            dimension_semantics=("parallel",),
        ),
    )(input, W, b2)
    return (logits, probs)


# speedup vs baseline: 2.4225x; 1.5828x over previous
import jax
import jax.numpy as jnp
from jax.experimental import pallas as pl
from jax.experimental.pallas import tpu as pltpu


def _router_kernel(x_ref, w_ref, b_ref, logits_ref, probs_ref):
    x = x_ref[...]
    w = w_ref[...]
    logits = jax.lax.dot_general(
        w, x,
        dimension_numbers=(((1,), (1,)), ((), ())),
        preferred_element_type=jnp.float32,
    ) + b_ref[...]
    logits_ref[...] = logits
    m = jnp.max(logits, axis=0, keepdims=True)
    e = jnp.exp(logits - m)
    probs_ref[...] = e / jnp.sum(e, axis=0, keepdims=True)


def kernel(input, W, b):
    n, d = input.shape
    num_experts = W.shape[0]
    tile = 1024
    grid = (n // tile,)
    b2 = b.reshape(num_experts, 1)
    logits_t, probs_t = pl.pallas_call(
        _router_kernel,
        grid=grid,
        in_specs=[
            pl.BlockSpec((tile, d), lambda i: (i, 0)),
            pl.BlockSpec((num_experts, d), lambda i: (0, 0)),
            pl.BlockSpec((num_experts, 1), lambda i: (0, 0)),
        ],
        out_specs=[
            pl.BlockSpec((num_experts, tile), lambda i: (0, i)),
            pl.BlockSpec((num_experts, tile), lambda i: (0, i)),
        ],
        out_shape=[
            jax.ShapeDtypeStruct((num_experts, n), jnp.float32),
            jax.ShapeDtypeStruct((num_experts, n), jnp.float32),
        ],
        compiler_params=pltpu.CompilerParams(
            dimension_semantics=("parallel",),
        ),
    )(input, W, b2)
    return (logits_t.T, probs_t.T)


# speedup vs baseline: 2.9381x; 1.2129x over previous
import jax
import jax.numpy as jnp
from jax.experimental import pallas as pl
from jax.experimental.pallas import tpu as pltpu


def _router_kernel(x_ref, w_ref, b_ref, logits_ref, probs_ref):
    x = x_ref[...]
    w = w_ref[...]
    logits = jax.lax.dot_general(
        w, x,
        dimension_numbers=(((1,), (1,)), ((), ())),
        preferred_element_type=jnp.float32,
    ) + b_ref[...]
    logits_ref[...] = logits
    m = jnp.max(logits, axis=0, keepdims=True)
    e = jnp.exp(logits - m)
    probs_ref[...] = e / jnp.sum(e, axis=0, keepdims=True)


def kernel(input, W, b):
    n, d = input.shape
    num_experts = W.shape[0]
    tile = 2048
    grid = (n // tile,)
    b2 = b.reshape(num_experts, 1)
    logits_t, probs_t = pl.pallas_call(
        _router_kernel,
        grid=grid,
        in_specs=[
            pl.BlockSpec((tile, d), lambda i: (i, 0)),
            pl.BlockSpec((num_experts, d), lambda i: (0, 0)),
            pl.BlockSpec((num_experts, 1), lambda i: (0, 0)),
        ],
        out_specs=[
            pl.BlockSpec((num_experts, tile), lambda i: (0, i)),
            pl.BlockSpec((num_experts, tile), lambda i: (0, i)),
        ],
        out_shape=[
            jax.ShapeDtypeStruct((num_experts, n), jnp.float32),
            jax.ShapeDtypeStruct((num_experts, n), jnp.float32),
        ],
        compiler_params=pltpu.CompilerParams(
            dimension_semantics=("parallel",),
        ),
    )(input, W, b2)
    return (logits_t.T, probs_t.T)


# speedup vs baseline: 3.0061x; 1.0231x over previous
import jax
import jax.numpy as jnp
from jax.experimental import pallas as pl
from jax.experimental.pallas import tpu as pltpu


def _router_kernel(x_ref, w_ref, b_ref, logits_ref, probs_ref):
    x = x_ref[...]
    w = w_ref[...]
    logits = jax.lax.dot_general(
        w, x,
        dimension_numbers=(((1,), (1,)), ((), ())),
        preferred_element_type=jnp.float32,
    ) + b_ref[...]
    logits_ref[...] = logits
    m = jnp.max(logits, axis=0, keepdims=True)
    e = jnp.exp(logits - m)
    probs_ref[...] = e / jnp.sum(e, axis=0, keepdims=True)


def kernel(input, W, b):
    n, d = input.shape
    num_experts = W.shape[0]
    tile = 4096
    grid = (n // tile,)
    b2 = b.reshape(num_experts, 1)
    logits_t, probs_t = pl.pallas_call(
        _router_kernel,
        grid=grid,
        in_specs=[
            pl.BlockSpec((tile, d), lambda i: (i, 0)),
            pl.BlockSpec((num_experts, d), lambda i: (0, 0)),
            pl.BlockSpec((num_experts, 1), lambda i: (0, 0)),
        ],
        out_specs=[
            pl.BlockSpec((num_experts, tile), lambda i: (0, i)),
            pl.BlockSpec((num_experts, tile), lambda i: (0, i)),
        ],
        out_shape=[
            jax.ShapeDtypeStruct((num_experts, n), jnp.float32),
            jax.ShapeDtypeStruct((num_experts, n), jnp.float32),
        ],
        compiler_params=pltpu.CompilerParams(
            dimension_semantics=("parallel",),
        ),
    )(input, W, b2)
    return (logits_t.T, probs_t.T)
